# baseline (speedup 1.0000x reference)
import functools
import math

import jax
import jax.numpy as jnp
import numpy as np
from jax import lax
from jax.experimental import pallas as pl
from jax.experimental.pallas import tpu as pltpu
from jax.experimental.pallas import tpu_sc as plsc

D = 512
B = 4
S = 2048
NFLAT = B * S
SCALE = math.sqrt(D)

NC, NS, L = 2, 16, 16
NW = NC * NS
POS_PER_W = S // NW
PH = 8
NPHASE = POS_PER_W // PH
NSET = 3


def _positional_encoding() -> np.ndarray:
    position = np.arange(S, dtype=np.float32)[:, None]
    div_term = np.exp(
        np.arange(0, D, 2, dtype=np.float32) * (-math.log(10000.0) / D)
    )
    pe = np.zeros((S, D), dtype=np.float32)
    pe[:, 0::2] = np.sin(position * div_term)
    pe[:, 1::2] = np.cos(position * div_term)
    return pe


_PE_F32 = _positional_encoding()


def _make_kernel():
    mesh = plsc.VectorSubcoreMesh(core_axis_name="c", subcore_axis_name="s")

    @functools.partial(
        pl.kernel,
        mesh=mesh,
        out_type=jax.ShapeDtypeStruct((NFLAT, D), jnp.float32),
        scratch_types=(
            [pltpu.VMEM((B, POS_PER_W), jnp.int32),
             pltpu.VMEM((POS_PER_W, D), jnp.float32)]
            + [pltpu.VMEM((PH, D), jnp.float32)] * (NSET * B)
            + [pltpu.SemaphoreType.DMA] * (2 + 2 * NSET)
        ),
    )
    def emb(x_hbm, table_hbm, pe_hbm, out_hbm, idx_v, pe_v, *bufs_and_sems):
        bufs = bufs_and_sems[:NSET * B]
        isem, psem = bufs_and_sems[NSET * B:NSET * B + 2]
        gsem = bufs_and_sems[NSET * B + 2:NSET * B + 2 + NSET]
        osem = bufs_and_sems[NSET * B + 2 + NSET:]
        sets = [bufs[s * B:(s + 1) * B] for s in range(NSET)]

        wid = lax.axis_index("s") * NC + lax.axis_index("c")
        pos0 = wid * POS_PER_W

        i_h = [
            pltpu.async_copy(
                x_hbm.at[b, pl.ds(pos0, POS_PER_W)], idx_v.at[b], isem)
            for b in range(B)
        ]
        p_h = pltpu.async_copy(pe_hbm.at[pl.ds(pos0, POS_PER_W)], pe_v, psem)
        for h in i_h:
            h.wait()

        g_h = [None] * NSET
        o_h = [None] * NSET

        def issue_gathers(p):
            s = p % NSET
            g_h[s] = [
                pltpu.async_copy(
                    table_hbm.at[idx_v.at[b, pl.ds(p * PH, PH)]],
                    sets[s][b], gsem[s])
                for b in range(B)
            ]

        def issue_outs(p):
            s = p % NSET
            o_h[s] = [
                pltpu.async_copy(
                    sets[s][b],
                    out_hbm.at[pl.ds(b * S + pos0 + p * PH, PH)],
                    osem[s])
                for b in range(B)
            ]

        issue_gathers(0)
        issue_gathers(1)

        for p in range(NPHASE):
            s = p % NSET
            if p + 2 < NPHASE:
                s2 = (p + 2) % NSET
                if o_h[s2] is not None:
                    for h in o_h[s2]:
                        h.wait()
                issue_gathers(p + 2)
            for h in g_h[s]:
                h.wait()
            if p == 0:
                p_h.wait()

            cur = sets[s]
            pe_base = p * PH

            def row(r, carry, cur=cur, pe_base=pe_base):
                for c in range(D // L):
                    sl = pl.ds(c * L, L)
                    pev = pe_v[pe_base + r, sl]
                    for b in range(B):
                        cur[b][r, sl] = cur[b][r, sl] * SCALE + pev
                return carry

            lax.fori_loop(0, PH, row, 0)
            issue_outs(p)
        for s in range(NSET):
            if o_h[s] is not None:
                for h in o_h[s]:
                    h.wait()

    return emb


_emb = _make_kernel()


def kernel(x, table):
    pe = jnp.asarray(_PE_F32)
    out = _emb(x, table, pe)
    return out.reshape(B, S, D)
